# baseline (speedup 1.0000x reference)
import functools

import jax
import jax.numpy as jnp
from jax import lax
from jax.experimental import pallas as pl
from jax.experimental.pallas import tpu as pltpu
from jax.experimental.pallas import tpu_sc as plsc

HIDDEN = 64
BATCH = 4096
SEQ = 200
TOTAL = BATCH * SEQ
PAIR_ROWS = 500000
NUM_WORKERS = 32
PER_WORKER = TOTAL // NUM_WORKERS
CHUNK = 128
NCHUNK = PER_WORKER // CHUNK

_mesh = plsc.VectorSubcoreMesh(core_axis_name="c", subcore_axis_name="s")


@functools.partial(
    pl.kernel,
    mesh=_mesh,
    out_type=jax.ShapeDtypeStruct((TOTAL, 2 * HIDDEN), jnp.float32),
    scratch_types=[
        pltpu.VMEM((NCHUNK, CHUNK), jnp.int32),
        pltpu.VMEM((2, CHUNK, 2 * HIDDEN), jnp.float32),
        pltpu.SemaphoreType.DMA,
        pltpu.SemaphoreType.DMA,
    ],
    compiler_params=pltpu.CompilerParams(needs_layout_passes=False),
)
def _gather_kernel(idx_hbm, pairs_hbm, out_hbm, idx_v, rows_v, gsem, ssem):
    wid = lax.axis_index("s") * 2 + lax.axis_index("c")
    base = wid * PER_WORKER

    pltpu.sync_copy(idx_hbm.at[wid], idx_v)

    def start_gather(i):
        pltpu.async_copy(pairs_hbm.at[idx_v.at[i]], rows_v.at[i % 2], gsem)

    def wait_gather():
        pltpu.make_async_copy(
            pairs_hbm.at[idx_v.at[0]], rows_v.at[0], gsem).wait()

    def start_store(i):
        pltpu.async_copy(
            rows_v.at[i % 2], out_hbm.at[pl.ds(base + i * CHUNK, CHUNK)], ssem)

    def wait_store():
        pltpu.make_async_copy(
            rows_v.at[0], out_hbm.at[pl.ds(base, CHUNK)], ssem).wait()

    start_gather(0)

    def body(i, carry):
        @pl.when(i >= 1)
        def _():
            wait_store()

        @pl.when(i + 1 < NCHUNK)
        def _():
            start_gather(i + 1)

        wait_gather()
        start_store(i)
        return carry

    lax.fori_loop(0, NCHUNK, body, 0)
    wait_store()


def kernel(input_ids, word_table):
    ids32 = input_ids.astype(jnp.int32)
    pair_idx = (ids32 >> 1).reshape(NUM_WORKERS, NCHUNK, CHUNK)
    pairs = jnp.concatenate(
        [word_table[0::2], word_table[1::2]], axis=1)
    out2 = _gather_kernel(pair_idx, pairs)
    out3 = out2.reshape(BATCH, SEQ, 2 * HIDDEN)
    par3 = (ids32 & 1)[:, :, None]
    return jnp.where(par3 == 1, out3[:, :, HIDDEN:], out3[:, :, :HIDDEN])
